# Initial kernel scaffold; baseline (speedup 1.0000x reference)
import jax
import jax.numpy as jnp
from jax.experimental import pallas as pl


def kernel(x, emb_word, emb_bi, emb_tri, W1, b1, W2, b2):
    raise NotImplementedError("write your pallas kernel here")


# speedup vs baseline: 1.3161x; 1.3161x over previous
import functools

import jax
import jax.numpy as jnp
from jax import lax
from jax.experimental import pallas as pl
from jax.experimental.pallas import tpu as pltpu
from jax.experimental.pallas import tpu_sc as plsc

VOCAB = 100000
NGRAM = 200000
EMBED = 300
HIDDEN = 256
NCLASS = 100
B = 4096
L = 50

NC = 2
NS = 16
NW = NC * NS
BPW = B // NW
G = 8
LANES = 16
NCH = 19
EMBED_P = NCH * LANES
IPW = BPW * L

GU = 16
LGU = GU.bit_length() - 1
UPR = (GU - 4 + EMBED + GU - 1) // GU
NPC = max(GU // LANES, 1)
NU = L * UPR


def _pool_body(x_hbm, ewg, ebg, etg, out_hbm, idx_v, gidx_v, rows_v, grp_v, sem):
    wid = lax.axis_index("s") * NC + lax.axis_index("c")
    base = wid * BPW
    for t in range(3):
        pltpu.sync_copy(
            x_hbm.at[t, pl.ds(base * L, IPW)], idx_v.at[pl.ds(t * IPW, IPW)]
        )

    iota = lax.iota(jnp.int32, LANES)
    uiota = iota * UPR

    def batch_body(b, carry):
        gb = lax.rem(b, G)
        for t, tabg, nunits in (
            (0, ewg, VOCAB * EMBED // GU),
            (1, ebg, NGRAM * EMBED // GU),
            (2, etg, NGRAM * EMBED // GU),
        ):
            tb = t * IPW + b * L
            for j0 in (0, 16, 32, L - LANES):
                vj = plsc.load_gather(idx_v, [jnp.full((LANES,), tb + j0, jnp.int32) + iota])
                u0 = (vj * EMBED) >> LGU
                addr0 = uiota + (j0 * UPR)
                for k in range(UPR):
                    plsc.store_scatter(
                        gidx_v, [addr0 + k], jnp.minimum(u0 + k, nunits - 1)
                    )
            pltpu.async_copy(tabg.at[gidx_v], rows_v, sem).wait()

            def j_body(j, accs):
                jf = jnp.full((LANES,), tb + j, jnp.int32)
                vj = plsc.load_gather(idx_v, [jf])
                o = (vj * EMBED) & (GU - 1)
                jbase = jnp.full((LANES,), j * UPR, jnp.int32)
                hbs = []
                los = []
                for i in range(NPC):
                    s = o + (iota + i * LANES)
                    hbs.append((s >> LGU) + jbase)
                    los.append(s & (GU - 1))
                new = []
                for c in range(NCH):
                    i = c % NPC
                    q = c // NPC
                    v = plsc.load_gather(rows_v, [hbs[i] + q, los[i]])
                    new.append(accs[c] + v)
                return tuple(new)

            accs = lax.fori_loop(
                0, L, j_body, tuple(jnp.zeros((LANES,), jnp.float32) for _ in range(NCH))
            )
            for c in range(NCH):
                grp_v[t, gb, pl.ds(c * LANES, LANES)] = accs[c] * (1.0 / L)

        @pl.when(gb == G - 1)
        def _flush():
            b0 = pl.multiple_of(base + b - (G - 1), G)
            for t in range(3):
                pltpu.sync_copy(grp_v.at[t], out_hbm.at[t, pl.ds(b0, G)])

        return carry

    lax.fori_loop(0, BPW, batch_body, 0)


_pool = functools.partial(
    pl.kernel,
    out_type=jax.ShapeDtypeStruct((3, B, EMBED_P), jnp.float32),
    mesh=plsc.VectorSubcoreMesh(
        core_axis_name="c", subcore_axis_name="s", num_cores=NC, num_subcores=NS
    ),
    scratch_types=[
        pltpu.VMEM((3 * IPW,), jnp.int32),
        pltpu.VMEM((NU,), jnp.int32),
        pltpu.VMEM((NU, GU), jnp.float32),
        pltpu.VMEM((3, G, EMBED_P), jnp.float32),
        pltpu.SemaphoreType.DMA,
    ],
    compiler_params=pltpu.CompilerParams(
        use_tc_tiling_on_sc=False, needs_layout_passes=False
    ),
)(_pool_body)


BB = 512


def _mlp_body(p_ref, w1_ref, b1_ref, w2_ref, b2_ref, o_ref):
    p = p_ref[...]
    h = jnp.dot(p[0, :, :EMBED], w1_ref[0:EMBED, :], preferred_element_type=jnp.float32)
    h = h + jnp.dot(
        p[1, :, :EMBED], w1_ref[EMBED : 2 * EMBED, :], preferred_element_type=jnp.float32
    )
    h = h + jnp.dot(
        p[2, :, :EMBED], w1_ref[2 * EMBED : 3 * EMBED, :], preferred_element_type=jnp.float32
    )
    h = jnp.maximum(h + b1_ref[...], 0.0)
    o_ref[...] = jnp.dot(h, w2_ref[...], preferred_element_type=jnp.float32) + b2_ref[...]


def _mlp(pooled, W1, b1, W2, b2):
    return pl.pallas_call(
        _mlp_body,
        grid=(B // BB,),
        in_specs=[
            pl.BlockSpec((3, BB, EMBED_P), lambda i: (0, i, 0)),
            pl.BlockSpec((3 * EMBED, HIDDEN), lambda i: (0, 0)),
            pl.BlockSpec((1, HIDDEN), lambda i: (0, 0)),
            pl.BlockSpec((HIDDEN, NCLASS), lambda i: (0, 0)),
            pl.BlockSpec((1, NCLASS), lambda i: (0, 0)),
        ],
        out_specs=pl.BlockSpec((BB, NCLASS), lambda i: (i, 0)),
        out_shape=jax.ShapeDtypeStruct((B, NCLASS), jnp.float32),
    )(pooled, W1, b1.reshape(1, HIDDEN), W2, b2.reshape(1, NCLASS))


@jax.jit
def kernel(x, emb_word, emb_bi, emb_tri, W1, b1, W2, b2):
    x2 = x.reshape(3, B * L)
    ewg = emb_word.reshape(-1, GU)
    ebg = emb_bi.reshape(-1, GU)
    etg = emb_tri.reshape(-1, GU)
    pooled = _pool(x2, ewg, ebg, etg)
    return _mlp(pooled, W1, b1, W2, b2)


# speedup vs baseline: 3.4348x; 2.6099x over previous
import functools

import jax
import jax.numpy as jnp
from jax import lax
from jax.experimental import pallas as pl
from jax.experimental.pallas import tpu as pltpu
from jax.experimental.pallas import tpu_sc as plsc

VOCAB = 100000
NGRAM = 200000
EMBED = 300
HIDDEN = 256
NCLASS = 100
B = 4096
L = 50

NC = 2
NS = 16
NW = NC * NS
BPW = B // NW
NOCT = BPW // 8
LANES = 16
EMBED_P = 304
TAIL0 = 256
OCTW = 3 * 400
HALF = 8 * OCTW
NR = 400


NSUB = NOCT * 9


def _pool_body(x_hbm, ew, eb, et, tw, tb_, tt, out_hbm,
               idx_v, idx2_v, rows_v, grp_v, sems):
    wid = lax.axis_index("s") * NC + lax.axis_index("c")
    base = wid * BPW

    tabs = (ew, eb, et)
    tails = (tw, tb_, tt)

    def fire(tau):
        par = lax.rem(tau, 2)
        oct_ = tau // 9
        r9 = lax.rem(tau, 9)
        t = r9 // 3
        p = lax.rem(r9, 3)
        ib = pl.multiple_of(lax.rem(oct_, NOCT // 2) * OCTW + t * 400, 16)
        i0 = pl.multiple_of(par * 512, 128)

        def m_body(m, carry):
            moff = pl.multiple_of(m * LANES, LANES)
            idx2_v[pl.ds(pl.multiple_of(i0 + moff, LANES), LANES)] = (
                idx_v[pl.ds(pl.multiple_of(ib + moff, LANES), LANES)]
            )
            return carry

        lax.fori_loop(0, NR // LANES, m_body, 0)
        idxref = idx2_v.at[pl.ds(i0, NR)]
        dst = rows_v.at[par]
        for t_ in range(3):
            @pl.when((t == t_) & (p < 2))
            def _():
                poff = pl.multiple_of(p * 128, 128)
                pltpu.async_copy(
                    tabs[t_].at[idxref, pl.ds(poff, 128)], dst, sems.at[par]
                )

            @pl.when((t == t_) & (p == 2))
            def _():
                pltpu.async_copy(tails[t_].at[idxref], dst, sems.at[par])

    def body(tau, carry):
        par = lax.rem(tau, 2)

        @pl.when(tau == NSUB // 2 - 1)
        def _stage2():
            src0 = pl.multiple_of(wid * (2 * HALF) + HALF, 128)
            pltpu.sync_copy(x_hbm.at[pl.ds(src0, HALF)], idx_v)

        @pl.when(tau < NSUB - 1)
        def _prefetch():
            fire(tau + 1)

        pltpu.make_async_copy(
            ew.at[pl.ds(0, NR), pl.ds(0, 128)], rows_v.at[par], sems.at[par]
        ).wait()

        r9 = lax.rem(tau, 9)
        t = r9 // 3
        p = lax.rem(r9, 3)

        def make_kbody(nchunks):
            def k_body(k, carry):
                def j_body(j, accs):
                    r = k * L + j
                    return tuple(
                        a + rows_v[par, r, pl.ds(cc * LANES, LANES)]
                        for cc, a in enumerate(accs)
                    )

                accs = lax.fori_loop(
                    0, L, j_body,
                    tuple(jnp.zeros((LANES,), jnp.float32) for _ in range(nchunks)),
                )
                for cc in range(nchunks):
                    off = pl.multiple_of((p * 8 + cc) * LANES, LANES)
                    grp_v[t, k, pl.ds(off, LANES)] = accs[cc] * (1.0 / L)
                return carry

            return k_body

        @pl.when(p < 2)
        def _acc8():
            lax.fori_loop(0, 8, make_kbody(8), 0)

        @pl.when(p == 2)
        def _acc3():
            lax.fori_loop(0, 8, make_kbody(3), 0)

        @pl.when(r9 == 8)
        def _flush():
            b0 = pl.multiple_of(base + (tau // 9) * 8, 8)
            for t_ in range(3):
                pltpu.sync_copy(grp_v.at[t_], out_hbm.at[t_, pl.ds(b0, 8)])

        return carry

    src0 = pl.multiple_of(wid * (2 * HALF), 128)
    pltpu.sync_copy(x_hbm.at[pl.ds(src0, HALF)], idx_v)
    fire(0)
    lax.fori_loop(0, NSUB, body, 0)


_pool = functools.partial(
    pl.kernel,
    out_type=jax.ShapeDtypeStruct((3, B, EMBED_P), jnp.float32),
    mesh=plsc.VectorSubcoreMesh(
        core_axis_name="c", subcore_axis_name="s", num_cores=NC, num_subcores=NS
    ),
    scratch_types=[
        pltpu.VMEM((HALF,), jnp.int32),
        pltpu.VMEM((1024,), jnp.int32),
        pltpu.VMEM((2, NR, 128), jnp.float32),
        pltpu.VMEM((3, 8, EMBED_P), jnp.float32),
        pltpu.SemaphoreType.DMA((2,)),
    ],
)(_pool_body)


BB = 512


def _mlp_body(p_ref, w1_ref, b1_ref, w2_ref, b2_ref, o_ref):
    p = p_ref[...]
    h = jnp.dot(p[0, :, :EMBED], w1_ref[0:EMBED, :], preferred_element_type=jnp.float32)
    h = h + jnp.dot(
        p[1, :, :EMBED], w1_ref[EMBED : 2 * EMBED, :], preferred_element_type=jnp.float32
    )
    h = h + jnp.dot(
        p[2, :, :EMBED], w1_ref[2 * EMBED : 3 * EMBED, :], preferred_element_type=jnp.float32
    )
    h = jnp.maximum(h + b1_ref[...], 0.0)
    o_ref[...] = jnp.dot(h, w2_ref[...], preferred_element_type=jnp.float32) + b2_ref[...]


def _mlp(pooled, W1, b1, W2, b2):
    return pl.pallas_call(
        _mlp_body,
        grid=(B // BB,),
        in_specs=[
            pl.BlockSpec((3, BB, EMBED_P), lambda i: (0, i, 0)),
            pl.BlockSpec((3 * EMBED, HIDDEN), lambda i: (0, 0)),
            pl.BlockSpec((1, HIDDEN), lambda i: (0, 0)),
            pl.BlockSpec((HIDDEN, NCLASS), lambda i: (0, 0)),
            pl.BlockSpec((1, NCLASS), lambda i: (0, 0)),
        ],
        out_specs=pl.BlockSpec((BB, NCLASS), lambda i: (i, 0)),
        out_shape=jax.ShapeDtypeStruct((B, NCLASS), jnp.float32),
    )(pooled, W1, b1.reshape(1, HIDDEN), W2, b2.reshape(1, NCLASS))


@jax.jit
def kernel(x, emb_word, emb_bi, emb_tri, W1, b1, W2, b2):
    xp = (
        x.transpose(1, 0, 2)
        .reshape(B // 8, 8, 3, L)
        .transpose(0, 2, 1, 3)
        .reshape(-1)
    )
    pad = ((0, 0), (0, 128 - (EMBED - TAIL0)))
    tw = jnp.pad(emb_word[:, TAIL0:EMBED], pad)
    tb_ = jnp.pad(emb_bi[:, TAIL0:EMBED], pad)
    tt = jnp.pad(emb_tri[:, TAIL0:EMBED], pad)
    pooled = _pool(xp, emb_word, emb_bi, emb_tri, tw, tb_, tt)
    return _mlp(pooled, W1, b1, W2, b2)


# speedup vs baseline: 3.4526x; 1.0052x over previous
import functools

import jax
import jax.numpy as jnp
from jax import lax
from jax.experimental import pallas as pl
from jax.experimental.pallas import tpu as pltpu
from jax.experimental.pallas import tpu_sc as plsc

VOCAB = 100000
NGRAM = 200000
EMBED = 300
HIDDEN = 256
NCLASS = 100
B = 4096
L = 50

NC = 2
NS = 16
NW = NC * NS
BPW = B // NW
NOCT = BPW // 8
LANES = 16
EMBED_P = 304
TAIL0 = 256
OCTW = 3 * 400
HALF = 8 * OCTW
NR = 400


NSUB = NOCT * 9


def _pool_body(x_hbm, ew, eb, et, tw, tb_, tt, out_hbm,
               idx_v, idx2_v, rows_v, grp_v, sems):
    wid = lax.axis_index("s") * NC + lax.axis_index("c")
    base = wid * BPW

    tabs = (ew, eb, et)
    tails = (tw, tb_, tt)

    def fire(tau):
        par = lax.rem(tau, 2)
        oct_ = tau // 9
        r9 = lax.rem(tau, 9)
        t = r9 // 3
        p = lax.rem(r9, 3)
        ib = pl.multiple_of(lax.rem(oct_, NOCT // 2) * OCTW + t * 400, 16)
        i0 = pl.multiple_of(par * 512, 128)

        def m_body(m, carry):
            moff = pl.multiple_of(m * LANES, LANES)
            idx2_v[pl.ds(pl.multiple_of(i0 + moff, LANES), LANES)] = (
                idx_v[pl.ds(pl.multiple_of(ib + moff, LANES), LANES)]
            )
            return carry

        lax.fori_loop(0, NR // LANES, m_body, 0)
        idxref = idx2_v.at[pl.ds(i0, NR)]
        dst = rows_v.at[par, pl.ds(0, NR)]
        for t_ in range(3):
            @pl.when((t == t_) & (p < 2))
            def _():
                poff = pl.multiple_of(p * 128, 128)
                pltpu.async_copy(
                    tabs[t_].at[idxref, pl.ds(poff, 128)], dst, sems.at[par]
                )

            @pl.when((t == t_) & (p == 2))
            def _():
                pltpu.async_copy(tails[t_].at[idxref], dst, sems.at[par])

    def body(tau, carry):
        par = lax.rem(tau, 2)

        @pl.when(tau == NSUB // 2 - 1)
        def _stage2():
            src0 = pl.multiple_of(wid * (2 * HALF) + HALF, 128)
            pltpu.sync_copy(x_hbm.at[pl.ds(src0, HALF)], idx_v)

        @pl.when(tau < NSUB - 1)
        def _prefetch():
            fire(tau + 1)

        pltpu.make_async_copy(
            ew.at[pl.ds(0, NR), pl.ds(0, 128)],
            rows_v.at[par, pl.ds(0, NR)],
            sems.at[par],
        ).wait()

        r9 = lax.rem(tau, 9)
        t = r9 // 3
        p = lax.rem(r9, 3)

        def make_kbody(roffs, woffs):
            def k_body(k, carry):
                def j_body(j, accs):
                    r = k * L + j
                    return tuple(
                        a + rows_v[par, r, pl.ds(ro, LANES)]
                        for ro, a in zip(roffs, accs)
                    )

                accs = lax.fori_loop(
                    0, L, j_body,
                    tuple(jnp.zeros((LANES,), jnp.float32) for _ in range(len(roffs))),
                )
                for wo, a in zip(woffs, accs):
                    grp_v[t, k, pl.ds(wo, LANES)] = a * (1.0 / L)
                return carry

            return k_body

        @pl.when(p < 2)
        def _acc8():
            roffs = tuple(cc * LANES for cc in range(8))
            woffs = tuple(pl.multiple_of(p * 128 + cc * LANES, LANES) for cc in range(8))
            lax.fori_loop(0, 8, make_kbody(roffs, woffs), 0)

        @pl.when(p == 2)
        def _acc3():
            lax.fori_loop(0, 8, make_kbody((0, 16, 32), (256, 272, 288)), 0)

        @pl.when(r9 == 8)
        def _flush():
            b0 = pl.multiple_of(base + (tau // 9) * 8, 8)
            for t_ in range(3):
                pltpu.sync_copy(grp_v.at[t_], out_hbm.at[t_, pl.ds(b0, 8)])

        return carry

    src0 = pl.multiple_of(wid * (2 * HALF), 128)
    pltpu.sync_copy(x_hbm.at[pl.ds(src0, HALF)], idx_v)
    fire(0)
    lax.fori_loop(0, NSUB, body, 0)


_pool = functools.partial(
    pl.kernel,
    out_type=jax.ShapeDtypeStruct((3, B, EMBED_P), jnp.float32),
    mesh=plsc.VectorSubcoreMesh(
        core_axis_name="c", subcore_axis_name="s", num_cores=NC, num_subcores=NS
    ),
    scratch_types=[
        pltpu.VMEM((HALF,), jnp.int32),
        pltpu.VMEM((1024,), jnp.int32),
        pltpu.VMEM((2, NR + 1, 128), jnp.float32),
        pltpu.VMEM((3, 8, EMBED_P), jnp.float32),
        pltpu.SemaphoreType.DMA((2,)),
    ],
)(_pool_body)


BLKT = 4000


def _tail_body(x_ref, o_ref):
    o_ref[:, 0 : EMBED - TAIL0] = x_ref[:, TAIL0:EMBED]


def _tailprep_one(tab, V):
    return pl.pallas_call(
        _tail_body,
        grid=(V // BLKT,),
        in_specs=[pl.BlockSpec((BLKT, EMBED), lambda i: (i, 0))],
        out_specs=pl.BlockSpec((BLKT, 128), lambda i: (i, 0)),
        out_shape=jax.ShapeDtypeStruct((V, 128), jnp.float32),
    )(tab)


def _tailprep(ew, eb, et):
    return (
        _tailprep_one(ew, VOCAB),
        _tailprep_one(eb, NGRAM),
        _tailprep_one(et, NGRAM),
    )


BB = 512


def _mlp_body(p_ref, w1_ref, b1_ref, w2_ref, b2_ref, o_ref):
    p = p_ref[...]
    h = jnp.dot(p[0, :, :EMBED], w1_ref[0:EMBED, :], preferred_element_type=jnp.float32)
    h = h + jnp.dot(
        p[1, :, :EMBED], w1_ref[EMBED : 2 * EMBED, :], preferred_element_type=jnp.float32
    )
    h = h + jnp.dot(
        p[2, :, :EMBED], w1_ref[2 * EMBED : 3 * EMBED, :], preferred_element_type=jnp.float32
    )
    h = jnp.maximum(h + b1_ref[...], 0.0)
    o_ref[...] = jnp.dot(h, w2_ref[...], preferred_element_type=jnp.float32) + b2_ref[...]


def _mlp(pooled, W1, b1, W2, b2):
    return pl.pallas_call(
        _mlp_body,
        grid=(B // BB,),
        in_specs=[
            pl.BlockSpec((3, BB, EMBED_P), lambda i: (0, i, 0)),
            pl.BlockSpec((3 * EMBED, HIDDEN), lambda i: (0, 0)),
            pl.BlockSpec((1, HIDDEN), lambda i: (0, 0)),
            pl.BlockSpec((HIDDEN, NCLASS), lambda i: (0, 0)),
            pl.BlockSpec((1, NCLASS), lambda i: (0, 0)),
        ],
        out_specs=pl.BlockSpec((BB, NCLASS), lambda i: (i, 0)),
        out_shape=jax.ShapeDtypeStruct((B, NCLASS), jnp.float32),
    )(pooled, W1, b1.reshape(1, HIDDEN), W2, b2.reshape(1, NCLASS))


@jax.jit
def kernel(x, emb_word, emb_bi, emb_tri, W1, b1, W2, b2):
    xp = (
        x.transpose(1, 0, 2)
        .reshape(B // 8, 8, 3, L)
        .transpose(0, 2, 1, 3)
        .reshape(-1)
    )
    tw, tb_, tt = _tailprep(emb_word, emb_bi, emb_tri)
    pooled = _pool(xp, emb_word, emb_bi, emb_tri, tw, tb_, tt)
    return _mlp(pooled, W1, b1, W2, b2)


# speedup vs baseline: 3.4783x; 1.0074x over previous
import functools

import jax
import jax.numpy as jnp
from jax import lax
from jax.experimental import pallas as pl
from jax.experimental.pallas import tpu as pltpu
from jax.experimental.pallas import tpu_sc as plsc

VOCAB = 100000
NGRAM = 200000
EMBED = 300
HIDDEN = 256
NCLASS = 100
B = 4096
L = 50

NC = 2
NS = 16
NW = NC * NS
BPW = B // NW
NOCT = BPW // 8
LANES = 16
EMBED_P = 304
TAIL0 = 256
OCTW = 3 * 400
HALF = 8 * OCTW
NR = 400


NSUB = NOCT * 9


def _pool_body(x_hbm, ew, eb, et, tw, tb_, tt, out_hbm,
               idx_v, idx2_v, rows_v, grp_v, sems):
    wid = lax.axis_index("s") * NC + lax.axis_index("c")
    base = wid * BPW

    tabs = (ew, eb, et)
    tails = (tw, tb_, tt)

    def fire(tau):
        par = lax.rem(tau, 2)
        oct_ = tau // 9
        r9 = lax.rem(tau, 9)
        t = r9 // 3
        p = lax.rem(r9, 3)
        ib = pl.multiple_of(t * (HALF // 3) + lax.rem(oct_, NOCT // 2) * 400, 16)
        i0 = pl.multiple_of(par * 512, 128)

        def m_body(m, carry):
            moff = pl.multiple_of(m * LANES, LANES)
            idx2_v[pl.ds(pl.multiple_of(i0 + moff, LANES), LANES)] = (
                idx_v[pl.ds(pl.multiple_of(ib + moff, LANES), LANES)]
            )
            return carry

        lax.fori_loop(0, NR // LANES, m_body, 0)
        idxref = idx2_v.at[pl.ds(i0, NR)]
        dst = rows_v.at[par, pl.ds(0, NR)]
        for t_ in range(3):
            @pl.when((t == t_) & (p < 2))
            def _():
                poff = pl.multiple_of(p * 128, 128)
                pltpu.async_copy(
                    tabs[t_].at[idxref, pl.ds(poff, 128)], dst, sems.at[par]
                )

            @pl.when((t == t_) & (p == 2))
            def _():
                pltpu.async_copy(tails[t_].at[idxref], dst, sems.at[par])

    def body(tau, carry):
        par = lax.rem(tau, 2)

        @pl.when(tau == NSUB // 2 - 1)
        def _stage2():
            for t_ in range(3):
                src0 = pl.multiple_of(
                    t_ * (B * L) + wid * (2 * HALF // 3) + HALF // 3, 128
                )
                pltpu.sync_copy(
                    x_hbm.at[pl.ds(src0, HALF // 3)],
                    idx_v.at[pl.ds(t_ * (HALF // 3), HALF // 3)],
                )

        @pl.when(tau < NSUB - 1)
        def _prefetch():
            fire(tau + 1)

        pltpu.make_async_copy(
            ew.at[pl.ds(0, NR), pl.ds(0, 128)],
            rows_v.at[par, pl.ds(0, NR)],
            sems.at[par],
        ).wait()

        r9 = lax.rem(tau, 9)
        t = r9 // 3
        p = lax.rem(r9, 3)

        def make_kbody(roffs, woffs):
            def k_body(k, carry):
                def j_body(j, accs):
                    r = k * L + j
                    return tuple(
                        a + rows_v[par, r, pl.ds(ro, LANES)]
                        for ro, a in zip(roffs, accs)
                    )

                accs = lax.fori_loop(
                    0, L, j_body,
                    tuple(jnp.zeros((LANES,), jnp.float32) for _ in range(len(roffs))),
                )
                for wo, a in zip(woffs, accs):
                    grp_v[t, k, pl.ds(wo, LANES)] = a * (1.0 / L)
                return carry

            return k_body

        @pl.when(p < 2)
        def _acc8():
            roffs = tuple(cc * LANES for cc in range(8))
            woffs = tuple(pl.multiple_of(p * 128 + cc * LANES, LANES) for cc in range(8))
            lax.fori_loop(0, 8, make_kbody(roffs, woffs), 0)

        @pl.when(p == 2)
        def _acc3():
            lax.fori_loop(0, 8, make_kbody((0, 16, 32), (256, 272, 288)), 0)

        @pl.when(r9 == 8)
        def _flush():
            b0 = pl.multiple_of(base + (tau // 9) * 8, 8)
            for t_ in range(3):
                pltpu.sync_copy(grp_v.at[t_], out_hbm.at[t_, pl.ds(b0, 8)])

        return carry

    for t_ in range(3):
        src0 = pl.multiple_of(t_ * (B * L) + wid * (2 * HALF // 3), 128)
        pltpu.sync_copy(
            x_hbm.at[pl.ds(src0, HALF // 3)],
            idx_v.at[pl.ds(t_ * (HALF // 3), HALF // 3)],
        )
    fire(0)
    lax.fori_loop(0, NSUB, body, 0)


_pool = functools.partial(
    pl.kernel,
    out_type=jax.ShapeDtypeStruct((3, B, EMBED_P), jnp.float32),
    mesh=plsc.VectorSubcoreMesh(
        core_axis_name="c", subcore_axis_name="s", num_cores=NC, num_subcores=NS
    ),
    scratch_types=[
        pltpu.VMEM((HALF,), jnp.int32),
        pltpu.VMEM((1024,), jnp.int32),
        pltpu.VMEM((2, NR + 1, 128), jnp.float32),
        pltpu.VMEM((3, 8, EMBED_P), jnp.float32),
        pltpu.SemaphoreType.DMA((2,)),
    ],
)(_pool_body)


BLKT = 4000


def _tail_body(x_ref, o_ref):
    o_ref[:, 0 : EMBED - TAIL0] = x_ref[:, TAIL0:EMBED]


def _tailprep_one(tab, V):
    return pl.pallas_call(
        _tail_body,
        grid=(V // BLKT,),
        in_specs=[pl.BlockSpec((BLKT, EMBED), lambda i: (i, 0))],
        out_specs=pl.BlockSpec((BLKT, 128), lambda i: (i, 0)),
        out_shape=jax.ShapeDtypeStruct((V, 128), jnp.float32),
    )(tab)


def _tailprep(ew, eb, et):
    return (
        _tailprep_one(ew, VOCAB),
        _tailprep_one(eb, NGRAM),
        _tailprep_one(et, NGRAM),
    )


BB = 512


def _mlp_body(p_ref, w1_ref, b1_ref, w2_ref, b2_ref, o_ref):
    p = p_ref[...]
    h = jnp.dot(p[0, :, :EMBED], w1_ref[0:EMBED, :], preferred_element_type=jnp.float32)
    h = h + jnp.dot(
        p[1, :, :EMBED], w1_ref[EMBED : 2 * EMBED, :], preferred_element_type=jnp.float32
    )
    h = h + jnp.dot(
        p[2, :, :EMBED], w1_ref[2 * EMBED : 3 * EMBED, :], preferred_element_type=jnp.float32
    )
    h = jnp.maximum(h + b1_ref[...], 0.0)
    o_ref[...] = jnp.dot(h, w2_ref[...], preferred_element_type=jnp.float32) + b2_ref[...]


def _mlp(pooled, W1, b1, W2, b2):
    return pl.pallas_call(
        _mlp_body,
        grid=(B // BB,),
        in_specs=[
            pl.BlockSpec((3, BB, EMBED_P), lambda i: (0, i, 0)),
            pl.BlockSpec((3 * EMBED, HIDDEN), lambda i: (0, 0)),
            pl.BlockSpec((1, HIDDEN), lambda i: (0, 0)),
            pl.BlockSpec((HIDDEN, NCLASS), lambda i: (0, 0)),
            pl.BlockSpec((1, NCLASS), lambda i: (0, 0)),
        ],
        out_specs=pl.BlockSpec((BB, NCLASS), lambda i: (i, 0)),
        out_shape=jax.ShapeDtypeStruct((B, NCLASS), jnp.float32),
    )(pooled, W1, b1.reshape(1, HIDDEN), W2, b2.reshape(1, NCLASS))


@jax.jit
def kernel(x, emb_word, emb_bi, emb_tri, W1, b1, W2, b2):
    xp = x.reshape(-1)
    tw, tb_, tt = _tailprep(emb_word, emb_bi, emb_tri)
    pooled = _pool(xp, emb_word, emb_bi, emb_tri, tw, tb_, tt)
    return _mlp(pooled, W1, b1, W2, b2)
